# baseline (speedup 1.0000x reference)
import functools

import jax
import jax.numpy as jnp
from jax import lax
from jax.experimental import pallas as pl
from jax.experimental.pallas import tpu as pltpu
from jax.experimental.pallas import tpu_sc as plsc

EMBED_DIM = 64
LANES = 16
N_WORKERS = 32
CHUNK = 128
ENTITY_AWARE_COFF = 0.001


TR_BLK = 512


def _tr_body(xt_hbm, out_hbm, in_v, tail_v, out_v, sem):
    n_nodes = xt_hbm.shape[1]
    n_full = n_nodes // TR_BLK
    tail = n_nodes - n_full * TR_BLK
    wid = lax.axis_index("s") * 2 + lax.axis_index("c")
    iota = lax.iota(jnp.int32, LANES)

    nblk = (n_full - 1 - wid) // N_WORKERS + 1

    def blk(t, carry):
        i0 = pl.multiple_of((wid + t * N_WORKERS) * TR_BLK, 128)
        pltpu.async_copy(xt_hbm.at[:, pl.ds(i0, TR_BLK)], in_v, sem).wait()

        def row(i, c):
            for jg in range(EMBED_DIM // LANES):
                vals = plsc.load_gather(
                    in_v, [jg * LANES + iota, jnp.full((LANES,), 0, jnp.int32) + i])
                out_v[pl.ds(i * EMBED_DIM + jg * LANES, LANES)] = vals
            return c

        lax.fori_loop(0, TR_BLK, row, 0)
        pltpu.sync_copy(out_v, out_hbm.at[pl.ds(i0 * EMBED_DIM, TR_BLK * EMBED_DIM)])
        return carry

    lax.fori_loop(0, nblk, blk, 0)

    if tail:
        @pl.when(wid == N_WORKERS - 1)
        def _():
            i0 = pl.multiple_of((wid - (N_WORKERS - 1)) + n_full * TR_BLK, 128)
            pltpu.async_copy(xt_hbm.at[:, pl.ds(i0, 128)], tail_v, sem).wait()

            def row(i, c):
                for jg in range(EMBED_DIM // LANES):
                    vals = plsc.load_gather(
                        tail_v,
                        [jg * LANES + iota, jnp.full((LANES,), 0, jnp.int32) + i])
                    out_v[pl.ds(i * EMBED_DIM + jg * LANES, LANES)] = vals
                return c

            lax.fori_loop(0, tail, row, 0)
            pltpu.sync_copy(
                out_v.at[pl.ds(0, tail * EMBED_DIM)],
                out_hbm.at[pl.ds(n_full * TR_BLK * EMBED_DIM, tail * EMBED_DIM)])


def _sc_linearize(x):
    n_nodes = x.shape[0]
    xt = x.T
    mesh = plsc.VectorSubcoreMesh(core_axis_name="c", subcore_axis_name="s")
    f = functools.partial(
        pl.kernel,
        out_type=jax.ShapeDtypeStruct((n_nodes * EMBED_DIM,), jnp.float32),
        mesh=mesh,
        scratch_types=[
            pltpu.VMEM((EMBED_DIM, TR_BLK), jnp.float32),
            pltpu.VMEM((EMBED_DIM, 128), jnp.float32),
            pltpu.VMEM((TR_BLK * EMBED_DIM,), jnp.float32),
            pltpu.SemaphoreType.DMA,
        ],
        compiler_params=pltpu.CompilerParams(
            use_tc_tiling_on_sc=True, needs_layout_passes=False,
            disable_bounds_checks=True),
    )(_tr_body)
    return f(xt)


def _sc_body(x_hbm, idx_hbm, pd_hbm, id_hbm, ud_hbm,
             idx_v, rows_v, pd_v, id_v, ud_v, z_v, sem):
    per_w = pd_v.shape[0]
    batch = pd_hbm.shape[0]
    n_chunks = per_w // CHUNK
    wid = lax.axis_index("s") * 2 + lax.axis_index("c")
    base_w = wid * per_w

    for k in range(7):
        pltpu.sync_copy(idx_hbm.at[pl.ds(k * batch + base_w, per_w)],
                        idx_v.at[pl.ds(k * per_w, per_w)])

    for c in range(n_chunks):
        cbase = c * CHUNK
        cps = [
            pltpu.async_copy(
                x_hbm.at[idx_v.at[pl.ds(k * per_w + cbase, CHUNK)]],
                rows_v.at[k], sem)
            for k in range(7)
        ]
        for cp in cps:
            cp.wait()

        def elem(e, carry):
            pd = None
            idp = None
            udp = None
            for j in range(EMBED_DIM // LANES):
                sl = pl.ds(j * LANES, LANES)
                uu = rows_v[0, e, sl]
                pp = rows_v[1, e, sl]
                nn = rows_v[2, e, sl]
                pe = rows_v[3, e, sl]
                ne = rows_v[4, e, sl]
                pu = rows_v[5, e, sl]
                nu = rows_v[6, e, sl]
                t_pd = uu * (pp - nn)
                a = pp - pe
                b = pp - ne
                t_id = a * a - b * b
                a2 = uu - pu
                b2 = uu - nu
                t_ud = a2 * a2 - b2 * b2
                pd = t_pd if pd is None else pd + t_pd
                idp = t_id if idp is None else idp + t_id
                udp = t_ud if udp is None else udp + t_ud
            pd_v[cbase + e, :] = pd
            id_v[cbase + e, :] = idp
            ud_v[cbase + e, :] = udp
            return carry

        lax.fori_loop(0, CHUNK, elem, 0)

    def group(g, carry):
        ids = g * LANES + lax.iota(jnp.int32, LANES)
        for t, part in enumerate((pd_v, id_v, ud_v)):
            z = None
            for l in range(LANES):
                col = plsc.load_gather(
                    part, [ids, jnp.full((LANES,), l, jnp.int32)])
                z = col if z is None else z + col
            z_v[t, pl.ds(g * LANES, LANES)] = z
        return carry

    lax.fori_loop(0, per_w // LANES, group, 0)

    pltpu.sync_copy(z_v.at[0], pd_hbm.at[pl.ds(base_w, per_w)])
    pltpu.sync_copy(z_v.at[1], id_hbm.at[pl.ds(base_w, per_w)])
    pltpu.sync_copy(z_v.at[2], ud_hbm.at[pl.ds(base_w, per_w)])


def _sc_partials(x, idx7):
    batch = idx7.shape[0] // 7
    per_w = batch // N_WORKERS
    mesh = plsc.VectorSubcoreMesh(core_axis_name="c", subcore_axis_name="s")
    out = jax.ShapeDtypeStruct((batch,), jnp.float32)
    f = functools.partial(
        pl.kernel,
        out_type=[out, out, out],
        mesh=mesh,
        scratch_types=[
            pltpu.VMEM((7 * per_w,), jnp.int32),
            pltpu.VMEM((7, CHUNK, EMBED_DIM), jnp.float32),
            pltpu.VMEM((per_w, LANES), jnp.float32),
            pltpu.VMEM((per_w, LANES), jnp.float32),
            pltpu.VMEM((per_w, LANES), jnp.float32),
            pltpu.VMEM((3, per_w), jnp.float32),
            pltpu.SemaphoreType.DMA,
        ],
        compiler_params=pltpu.CompilerParams(
            use_tc_tiling_on_sc=False, needs_layout_passes=False),
    )(_sc_body)
    return f(x, idx7)


def _tc_body(z_ref, zi_ref, zu_ref, mi_ref, mu_ref, out_ref):
    z = z_ref[...]
    zi = zi_ref[...] * mi_ref[...]
    zu = zu_ref[...] * mu_ref[...]

    def nls(t):
        mt = jnp.maximum(-t, 0.0)
        return mt + jnp.log(jnp.exp(-t - mt) + jnp.exp(-mt))

    cf = jnp.sum(nls(z))
    reg = jnp.sum(nls(zi)) + jnp.sum(nls(zu))
    out_ref[0, 0] = cf + ENTITY_AWARE_COFF * reg


def _tc_finish(z, zi, zu, mi, mu):
    batch = z.shape[0]
    rows = 128
    cols = batch // rows
    out = pl.pallas_call(
        _tc_body,
        out_shape=jax.ShapeDtypeStruct((1, 1), jnp.float32),
        out_specs=pl.BlockSpec(memory_space=pltpu.SMEM),
    )(z.reshape(rows, cols), zi.reshape(rows, cols), zu.reshape(rows, cols),
      mi.reshape(rows, cols), mu.reshape(rows, cols))
    return out[0, 0]


def kernel(x, pos_neg_pair_t):
    p = pos_neg_pair_t.astype(jnp.int32)
    cols = p.T
    idx7 = jnp.concatenate(
        [cols[0:5], cols[6:8]], axis=0).reshape(-1)
    mi = cols[5].astype(jnp.float32)
    mu = cols[8].astype(jnp.float32)
    x_flat = _sc_linearize(x)
    x_lin = x_flat.reshape(x.shape)
    z, zi, zu = _sc_partials(x_lin, idx7)
    return _tc_finish(z, zi, zu, mi, mu)


# speedup vs baseline: 1.1932x; 1.1932x over previous
import functools

import jax
import jax.numpy as jnp
from jax import lax
from jax.experimental import pallas as pl
from jax.experimental.pallas import tpu as pltpu
from jax.experimental.pallas import tpu_sc as plsc

EMBED_DIM = 64
LANES = 16
N_WORKERS = 32
CHUNK = 128
ENTITY_AWARE_COFF = 0.001


TR_BLK = 512


def _tr_body(xt_hbm, out_hbm, in_v, tail_v, out_v, sem):
    n_nodes = xt_hbm.shape[1]
    n_full = n_nodes // TR_BLK
    tail = n_nodes - n_full * TR_BLK
    wid = lax.axis_index("s") * 2 + lax.axis_index("c")
    iota = lax.iota(jnp.int32, LANES)

    nblk = (n_full - 1 - wid) // N_WORKERS + 1

    def transpose_groups(buf, n_groups):
        def grp(g, c):
            base = g * LANES
            rowoff = (base + iota) * EMBED_DIM
            for j in range(EMBED_DIM):
                vals = buf[j, pl.ds(base, LANES)]
                plsc.store_scatter(out_v, [rowoff + j], vals)
            return c

        lax.fori_loop(0, n_groups, grp, 0)

    def blk(t, carry):
        i0 = pl.multiple_of((wid + t * N_WORKERS) * TR_BLK, 128)
        pltpu.async_copy(xt_hbm.at[:, pl.ds(i0, TR_BLK)], in_v, sem).wait()
        transpose_groups(in_v, TR_BLK // LANES)
        pltpu.sync_copy(out_v, out_hbm.at[pl.ds(i0 * EMBED_DIM, TR_BLK * EMBED_DIM)])
        return carry

    lax.fori_loop(0, nblk, blk, 0)

    if tail:
        @pl.when(wid == N_WORKERS - 1)
        def _():
            i0 = pl.multiple_of((wid - (N_WORKERS - 1)) + n_full * TR_BLK, 128)
            pltpu.async_copy(xt_hbm.at[:, pl.ds(i0, 128)], tail_v, sem).wait()
            transpose_groups(tail_v, tail // LANES)
            pltpu.sync_copy(
                out_v.at[pl.ds(0, tail * EMBED_DIM)],
                out_hbm.at[pl.ds(n_full * TR_BLK * EMBED_DIM, tail * EMBED_DIM)])


def _sc_linearize(x):
    n_nodes = x.shape[0]
    xt = x.T
    mesh = plsc.VectorSubcoreMesh(core_axis_name="c", subcore_axis_name="s")
    f = functools.partial(
        pl.kernel,
        out_type=jax.ShapeDtypeStruct((n_nodes * EMBED_DIM,), jnp.float32),
        mesh=mesh,
        scratch_types=[
            pltpu.VMEM((EMBED_DIM, TR_BLK), jnp.float32),
            pltpu.VMEM((EMBED_DIM, 128), jnp.float32),
            pltpu.VMEM((TR_BLK * EMBED_DIM,), jnp.float32),
            pltpu.SemaphoreType.DMA,
        ],
        compiler_params=pltpu.CompilerParams(
            use_tc_tiling_on_sc=True, needs_layout_passes=False,
            disable_bounds_checks=True),
    )(_tr_body)
    return f(xt)


def _sc_body(x_hbm, idx_hbm, pd_hbm, id_hbm, ud_hbm,
             idx_v, rows_v, pd_v, id_v, ud_v, z_v, sem):
    per_w = pd_v.shape[0]
    batch = pd_hbm.shape[0]
    n_chunks = per_w // CHUNK
    wid = lax.axis_index("s") * 2 + lax.axis_index("c")
    base_w = wid * per_w

    for k in range(7):
        pltpu.sync_copy(idx_hbm.at[pl.ds(k * batch + base_w, per_w)],
                        idx_v.at[pl.ds(k * per_w, per_w)])

    for c in range(n_chunks):
        cbase = c * CHUNK
        cps = [
            pltpu.async_copy(
                x_hbm.at[idx_v.at[pl.ds(k * per_w + cbase, CHUNK)]],
                rows_v.at[k], sem)
            for k in range(7)
        ]
        for cp in cps:
            cp.wait()

        def elem(e, carry):
            pd = None
            idp = None
            udp = None
            for j in range(EMBED_DIM // LANES):
                sl = pl.ds(j * LANES, LANES)
                uu = rows_v[0, e, sl]
                pp = rows_v[1, e, sl]
                nn = rows_v[2, e, sl]
                pe = rows_v[3, e, sl]
                ne = rows_v[4, e, sl]
                pu = rows_v[5, e, sl]
                nu = rows_v[6, e, sl]
                t_pd = uu * (pp - nn)
                a = pp - pe
                b = pp - ne
                t_id = a * a - b * b
                a2 = uu - pu
                b2 = uu - nu
                t_ud = a2 * a2 - b2 * b2
                pd = t_pd if pd is None else pd + t_pd
                idp = t_id if idp is None else idp + t_id
                udp = t_ud if udp is None else udp + t_ud
            pd_v[cbase + e, :] = pd
            id_v[cbase + e, :] = idp
            ud_v[cbase + e, :] = udp
            return carry

        lax.fori_loop(0, CHUNK, elem, 0)

    def group(g, carry):
        ids = g * LANES + lax.iota(jnp.int32, LANES)
        for t, part in enumerate((pd_v, id_v, ud_v)):
            z = None
            for l in range(LANES):
                col = plsc.load_gather(
                    part, [ids, jnp.full((LANES,), l, jnp.int32)])
                z = col if z is None else z + col
            z_v[t, pl.ds(g * LANES, LANES)] = z
        return carry

    lax.fori_loop(0, per_w // LANES, group, 0)

    pltpu.sync_copy(z_v.at[0], pd_hbm.at[pl.ds(base_w, per_w)])
    pltpu.sync_copy(z_v.at[1], id_hbm.at[pl.ds(base_w, per_w)])
    pltpu.sync_copy(z_v.at[2], ud_hbm.at[pl.ds(base_w, per_w)])


def _sc_partials(x, idx7):
    batch = idx7.shape[0] // 7
    per_w = batch // N_WORKERS
    mesh = plsc.VectorSubcoreMesh(core_axis_name="c", subcore_axis_name="s")
    out = jax.ShapeDtypeStruct((batch,), jnp.float32)
    f = functools.partial(
        pl.kernel,
        out_type=[out, out, out],
        mesh=mesh,
        scratch_types=[
            pltpu.VMEM((7 * per_w,), jnp.int32),
            pltpu.VMEM((7, CHUNK, EMBED_DIM), jnp.float32),
            pltpu.VMEM((per_w, LANES), jnp.float32),
            pltpu.VMEM((per_w, LANES), jnp.float32),
            pltpu.VMEM((per_w, LANES), jnp.float32),
            pltpu.VMEM((3, per_w), jnp.float32),
            pltpu.SemaphoreType.DMA,
        ],
        compiler_params=pltpu.CompilerParams(
            use_tc_tiling_on_sc=False, needs_layout_passes=False),
    )(_sc_body)
    return f(x, idx7)


def _tc_body(z_ref, zi_ref, zu_ref, mi_ref, mu_ref, out_ref):
    z = z_ref[...]
    zi = zi_ref[...] * mi_ref[...]
    zu = zu_ref[...] * mu_ref[...]

    def nls(t):
        mt = jnp.maximum(-t, 0.0)
        return mt + jnp.log(jnp.exp(-t - mt) + jnp.exp(-mt))

    cf = jnp.sum(nls(z))
    reg = jnp.sum(nls(zi)) + jnp.sum(nls(zu))
    out_ref[0, 0] = cf + ENTITY_AWARE_COFF * reg


def _tc_finish(z, zi, zu, mi, mu):
    batch = z.shape[0]
    rows = 128
    cols = batch // rows
    out = pl.pallas_call(
        _tc_body,
        out_shape=jax.ShapeDtypeStruct((1, 1), jnp.float32),
        out_specs=pl.BlockSpec(memory_space=pltpu.SMEM),
    )(z.reshape(rows, cols), zi.reshape(rows, cols), zu.reshape(rows, cols),
      mi.reshape(rows, cols), mu.reshape(rows, cols))
    return out[0, 0]


def kernel(x, pos_neg_pair_t):
    p = pos_neg_pair_t.astype(jnp.int32)
    cols = p.T
    idx7 = jnp.concatenate(
        [cols[0:5], cols[6:8]], axis=0).reshape(-1)
    mi = cols[5].astype(jnp.float32)
    mu = cols[8].astype(jnp.float32)
    x_flat = _sc_linearize(x)
    x_lin = x_flat.reshape(x.shape)
    z, zi, zu = _sc_partials(x_lin, idx7)
    return _tc_finish(z, zi, zu, mi, mu)


# speedup vs baseline: 1.5621x; 1.3092x over previous
import functools

import jax
import jax.numpy as jnp
from jax import lax
from jax.experimental import pallas as pl
from jax.experimental.pallas import tpu as pltpu
from jax.experimental.pallas import tpu_sc as plsc

EMBED_DIM = 64
LANES = 16
N_WORKERS = 32
CHUNK = 128
ENTITY_AWARE_COFF = 0.001


TR_BLK = 512


def _tr_body(xt_hbm, out_hbm, in_v, tail_v, out_v, sem):
    n_nodes = xt_hbm.shape[1]
    n_full = n_nodes // TR_BLK
    tail = n_nodes - n_full * TR_BLK
    wid = lax.axis_index("s") * 2 + lax.axis_index("c")
    iota = lax.iota(jnp.int32, LANES)

    nblk = (n_full - 1 - wid) // N_WORKERS + 1

    def transpose_groups(buf, n_groups):
        def grp(g, c):
            base = g * LANES
            rowoff = (base + iota) * EMBED_DIM
            look = 8
            vals = {j: buf[j, pl.ds(base, LANES)] for j in range(look)}
            for j in range(EMBED_DIM):
                if j + look < EMBED_DIM:
                    vals[j + look] = buf[j + look, pl.ds(base, LANES)]
                plsc.store_scatter(out_v, [rowoff + j], vals[j])
            return c

        lax.fori_loop(0, n_groups, grp, 0)

    def blk(t, carry):
        i0 = pl.multiple_of((wid + t * N_WORKERS) * TR_BLK, 128)
        pltpu.async_copy(xt_hbm.at[:, pl.ds(i0, TR_BLK)], in_v, sem).wait()
        transpose_groups(in_v, TR_BLK // LANES)
        pltpu.sync_copy(out_v, out_hbm.at[pl.ds(i0 * EMBED_DIM, TR_BLK * EMBED_DIM)])
        return carry

    lax.fori_loop(0, nblk, blk, 0)

    if tail:
        @pl.when(wid == N_WORKERS - 1)
        def _():
            i0 = pl.multiple_of((wid - (N_WORKERS - 1)) + n_full * TR_BLK, 128)
            pltpu.async_copy(xt_hbm.at[:, pl.ds(i0, 128)], tail_v, sem).wait()
            transpose_groups(tail_v, tail // LANES)
            pltpu.sync_copy(
                out_v.at[pl.ds(0, tail * EMBED_DIM)],
                out_hbm.at[pl.ds(n_full * TR_BLK * EMBED_DIM, tail * EMBED_DIM)])


def _sc_linearize(x):
    n_nodes = x.shape[0]
    xt = x.T
    mesh = plsc.VectorSubcoreMesh(core_axis_name="c", subcore_axis_name="s")
    f = functools.partial(
        pl.kernel,
        out_type=jax.ShapeDtypeStruct((n_nodes * EMBED_DIM,), jnp.float32),
        mesh=mesh,
        scratch_types=[
            pltpu.VMEM((EMBED_DIM, TR_BLK), jnp.float32),
            pltpu.VMEM((EMBED_DIM, 128), jnp.float32),
            pltpu.VMEM((TR_BLK * EMBED_DIM,), jnp.float32),
            pltpu.SemaphoreType.DMA,
        ],
        compiler_params=pltpu.CompilerParams(
            use_tc_tiling_on_sc=True, needs_layout_passes=False,
            disable_bounds_checks=True),
    )(_tr_body)
    return f(xt)


def _sc_body(x_hbm, idx_hbm, pd_hbm, id_hbm, ud_hbm,
             idx_v, rows_v, pd_v, id_v, ud_v, z_v, sem):
    per_w = pd_v.shape[0]
    batch = pd_hbm.shape[0]
    n_chunks = per_w // CHUNK
    wid = lax.axis_index("s") * 2 + lax.axis_index("c")
    base_w = wid * per_w

    for k in range(7):
        pltpu.sync_copy(idx_hbm.at[pl.ds(k * batch + base_w, per_w)],
                        idx_v.at[pl.ds(k * per_w, per_w)])

    for c in range(n_chunks):
        cbase = c * CHUNK
        cps = [
            pltpu.async_copy(
                x_hbm.at[idx_v.at[pl.ds(k * per_w + cbase, CHUNK)]],
                rows_v.at[k], sem)
            for k in range(7)
        ]
        for cp in cps:
            cp.wait()

        def elem(e, carry):
            pd = None
            idp = None
            udp = None
            for j in range(EMBED_DIM // LANES):
                sl = pl.ds(j * LANES, LANES)
                uu = rows_v[0, e, sl]
                pp = rows_v[1, e, sl]
                nn = rows_v[2, e, sl]
                pe = rows_v[3, e, sl]
                ne = rows_v[4, e, sl]
                pu = rows_v[5, e, sl]
                nu = rows_v[6, e, sl]
                t_pd = uu * (pp - nn)
                a = pp - pe
                b = pp - ne
                t_id = a * a - b * b
                a2 = uu - pu
                b2 = uu - nu
                t_ud = a2 * a2 - b2 * b2
                pd = t_pd if pd is None else pd + t_pd
                idp = t_id if idp is None else idp + t_id
                udp = t_ud if udp is None else udp + t_ud
            pd_v[cbase + e, :] = pd
            id_v[cbase + e, :] = idp
            ud_v[cbase + e, :] = udp
            return carry

        lax.fori_loop(0, CHUNK, elem, 0)

    def group(g, carry):
        ids = g * LANES + lax.iota(jnp.int32, LANES)
        for t, part in enumerate((pd_v, id_v, ud_v)):
            z = None
            for l in range(LANES):
                col = plsc.load_gather(
                    part, [ids, jnp.full((LANES,), l, jnp.int32)])
                z = col if z is None else z + col
            z_v[t, pl.ds(g * LANES, LANES)] = z
        return carry

    lax.fori_loop(0, per_w // LANES, group, 0)

    pltpu.sync_copy(z_v.at[0], pd_hbm.at[pl.ds(base_w, per_w)])
    pltpu.sync_copy(z_v.at[1], id_hbm.at[pl.ds(base_w, per_w)])
    pltpu.sync_copy(z_v.at[2], ud_hbm.at[pl.ds(base_w, per_w)])


def _sc_partials(x, idx7):
    batch = idx7.shape[0] // 7
    per_w = batch // N_WORKERS
    mesh = plsc.VectorSubcoreMesh(core_axis_name="c", subcore_axis_name="s")
    out = jax.ShapeDtypeStruct((batch,), jnp.float32)
    f = functools.partial(
        pl.kernel,
        out_type=[out, out, out],
        mesh=mesh,
        scratch_types=[
            pltpu.VMEM((7 * per_w,), jnp.int32),
            pltpu.VMEM((7, CHUNK, EMBED_DIM), jnp.float32),
            pltpu.VMEM((per_w, LANES), jnp.float32),
            pltpu.VMEM((per_w, LANES), jnp.float32),
            pltpu.VMEM((per_w, LANES), jnp.float32),
            pltpu.VMEM((3, per_w), jnp.float32),
            pltpu.SemaphoreType.DMA,
        ],
        compiler_params=pltpu.CompilerParams(
            use_tc_tiling_on_sc=False, needs_layout_passes=False),
    )(_sc_body)
    return f(x, idx7)


def _tc_body(z_ref, zi_ref, zu_ref, mi_ref, mu_ref, out_ref):
    z = z_ref[...]
    zi = zi_ref[...] * mi_ref[...]
    zu = zu_ref[...] * mu_ref[...]

    def nls(t):
        mt = jnp.maximum(-t, 0.0)
        return mt + jnp.log(jnp.exp(-t - mt) + jnp.exp(-mt))

    cf = jnp.sum(nls(z))
    reg = jnp.sum(nls(zi)) + jnp.sum(nls(zu))
    out_ref[0, 0] = cf + ENTITY_AWARE_COFF * reg


def _tc_finish(z, zi, zu, mi, mu):
    batch = z.shape[0]
    rows = 128
    cols = batch // rows
    out = pl.pallas_call(
        _tc_body,
        out_shape=jax.ShapeDtypeStruct((1, 1), jnp.float32),
        out_specs=pl.BlockSpec(memory_space=pltpu.SMEM),
    )(z.reshape(rows, cols), zi.reshape(rows, cols), zu.reshape(rows, cols),
      mi.reshape(rows, cols), mu.reshape(rows, cols))
    return out[0, 0]


def kernel(x, pos_neg_pair_t):
    p = pos_neg_pair_t.astype(jnp.int32)
    cols = p.T
    idx7 = jnp.concatenate(
        [cols[0:5], cols[6:8]], axis=0).reshape(-1)
    mi = cols[5].astype(jnp.float32)
    mu = cols[8].astype(jnp.float32)
    x_flat = _sc_linearize(x)
    x_lin = x_flat.reshape(x.shape)
    z, zi, zu = _sc_partials(x_lin, idx7)
    return _tc_finish(z, zi, zu, mi, mu)


# speedup vs baseline: 1.8770x; 1.2016x over previous
import functools

import jax
import jax.numpy as jnp
from jax import lax
from jax.experimental import pallas as pl
from jax.experimental.pallas import tpu as pltpu
from jax.experimental.pallas import tpu_sc as plsc

EMBED_DIM = 64
LANES = 16
N_WORKERS = 32
CHUNK = 128
ENTITY_AWARE_COFF = 0.001


TR_BLK = 512


def _tr_body(xt_hbm, out_hbm, in_v, tail_v, out_v, sem):
    n_nodes = xt_hbm.shape[1]
    n_full = n_nodes // TR_BLK
    tail = n_nodes - n_full * TR_BLK
    wid = lax.axis_index("s") * 2 + lax.axis_index("c")
    iota = lax.iota(jnp.int32, LANES)

    nblk = (n_full - 1 - wid) // N_WORKERS + 1

    rot = [(iota + s) & (LANES - 1) for s in range(LANES)]

    def transpose_groups(buf, n_groups):
        def grp(g, c):
            base = g * LANES
            colv = base + iota
            rowoff = colv * EMBED_DIM
            for jb in range(EMBED_DIM // LANES):
                for s in range(LANES):
                    rot_jb = rot[s] + jb * LANES
                    vals = plsc.load_gather(buf, [rot_jb, colv])
                    plsc.store_scatter(out_v, [rowoff + rot_jb], vals)
            return c

        lax.fori_loop(0, n_groups, grp, 0)

    def blk(t, carry):
        i0 = pl.multiple_of((wid + t * N_WORKERS) * TR_BLK, 128)
        pltpu.async_copy(xt_hbm.at[:, pl.ds(i0, TR_BLK)], in_v, sem).wait()
        transpose_groups(in_v, TR_BLK // LANES)
        pltpu.sync_copy(out_v, out_hbm.at[pl.ds(i0 * EMBED_DIM, TR_BLK * EMBED_DIM)])
        return carry

    lax.fori_loop(0, nblk, blk, 0)

    if tail:
        @pl.when(wid == N_WORKERS - 1)
        def _():
            i0 = pl.multiple_of((wid - (N_WORKERS - 1)) + n_full * TR_BLK, 128)
            pltpu.async_copy(xt_hbm.at[:, pl.ds(i0, 128)], tail_v, sem).wait()
            transpose_groups(tail_v, tail // LANES)
            pltpu.sync_copy(
                out_v.at[pl.ds(0, tail * EMBED_DIM)],
                out_hbm.at[pl.ds(n_full * TR_BLK * EMBED_DIM, tail * EMBED_DIM)])


def _sc_linearize(x):
    n_nodes = x.shape[0]
    xt = x.T
    mesh = plsc.VectorSubcoreMesh(core_axis_name="c", subcore_axis_name="s")
    f = functools.partial(
        pl.kernel,
        out_type=jax.ShapeDtypeStruct((n_nodes * EMBED_DIM,), jnp.float32),
        mesh=mesh,
        scratch_types=[
            pltpu.VMEM((EMBED_DIM, TR_BLK), jnp.float32),
            pltpu.VMEM((EMBED_DIM, 128), jnp.float32),
            pltpu.VMEM((TR_BLK * EMBED_DIM,), jnp.float32),
            pltpu.SemaphoreType.DMA,
        ],
        compiler_params=pltpu.CompilerParams(
            use_tc_tiling_on_sc=True, needs_layout_passes=False,
            disable_bounds_checks=True),
    )(_tr_body)
    return f(xt)


def _sc_body(x_hbm, idx_hbm, pd_hbm, id_hbm, ud_hbm,
             idx_v, rows_v, pd_v, id_v, ud_v, z_v, sem):
    per_w = pd_v.shape[0]
    batch = pd_hbm.shape[0]
    n_chunks = per_w // CHUNK
    wid = lax.axis_index("s") * 2 + lax.axis_index("c")
    base_w = wid * per_w

    for k in range(7):
        pltpu.sync_copy(idx_hbm.at[pl.ds(k * batch + base_w, per_w)],
                        idx_v.at[pl.ds(k * per_w, per_w)])

    for c in range(n_chunks):
        cbase = c * CHUNK
        cps = [
            pltpu.async_copy(
                x_hbm.at[idx_v.at[pl.ds(k * per_w + cbase, CHUNK)]],
                rows_v.at[k], sem)
            for k in range(7)
        ]
        for cp in cps:
            cp.wait()

        def elem(e, carry):
            pd = None
            idp = None
            udp = None
            for j in range(EMBED_DIM // LANES):
                sl = pl.ds(j * LANES, LANES)
                uu = rows_v[0, e, sl]
                pp = rows_v[1, e, sl]
                nn = rows_v[2, e, sl]
                pe = rows_v[3, e, sl]
                ne = rows_v[4, e, sl]
                pu = rows_v[5, e, sl]
                nu = rows_v[6, e, sl]
                t_pd = uu * (pp - nn)
                a = pp - pe
                b = pp - ne
                t_id = a * a - b * b
                a2 = uu - pu
                b2 = uu - nu
                t_ud = a2 * a2 - b2 * b2
                pd = t_pd if pd is None else pd + t_pd
                idp = t_id if idp is None else idp + t_id
                udp = t_ud if udp is None else udp + t_ud
            pd_v[cbase + e, :] = pd
            id_v[cbase + e, :] = idp
            ud_v[cbase + e, :] = udp
            return carry

        lax.fori_loop(0, CHUNK, elem, 0)

    def group(g, carry):
        ids = g * LANES + lax.iota(jnp.int32, LANES)
        for t, part in enumerate((pd_v, id_v, ud_v)):
            z = None
            for l in range(LANES):
                col = plsc.load_gather(
                    part, [ids, jnp.full((LANES,), l, jnp.int32)])
                z = col if z is None else z + col
            z_v[t, pl.ds(g * LANES, LANES)] = z
        return carry

    lax.fori_loop(0, per_w // LANES, group, 0)

    pltpu.sync_copy(z_v.at[0], pd_hbm.at[pl.ds(base_w, per_w)])
    pltpu.sync_copy(z_v.at[1], id_hbm.at[pl.ds(base_w, per_w)])
    pltpu.sync_copy(z_v.at[2], ud_hbm.at[pl.ds(base_w, per_w)])


def _sc_partials(x, idx7):
    batch = idx7.shape[0] // 7
    per_w = batch // N_WORKERS
    mesh = plsc.VectorSubcoreMesh(core_axis_name="c", subcore_axis_name="s")
    out = jax.ShapeDtypeStruct((batch,), jnp.float32)
    f = functools.partial(
        pl.kernel,
        out_type=[out, out, out],
        mesh=mesh,
        scratch_types=[
            pltpu.VMEM((7 * per_w,), jnp.int32),
            pltpu.VMEM((7, CHUNK, EMBED_DIM), jnp.float32),
            pltpu.VMEM((per_w, LANES), jnp.float32),
            pltpu.VMEM((per_w, LANES), jnp.float32),
            pltpu.VMEM((per_w, LANES), jnp.float32),
            pltpu.VMEM((3, per_w), jnp.float32),
            pltpu.SemaphoreType.DMA,
        ],
        compiler_params=pltpu.CompilerParams(
            use_tc_tiling_on_sc=False, needs_layout_passes=False),
    )(_sc_body)
    return f(x, idx7)


def _tc_body(z_ref, zi_ref, zu_ref, mi_ref, mu_ref, out_ref):
    z = z_ref[...]
    zi = zi_ref[...] * mi_ref[...]
    zu = zu_ref[...] * mu_ref[...]

    def nls(t):
        mt = jnp.maximum(-t, 0.0)
        return mt + jnp.log(jnp.exp(-t - mt) + jnp.exp(-mt))

    cf = jnp.sum(nls(z))
    reg = jnp.sum(nls(zi)) + jnp.sum(nls(zu))
    out_ref[0, 0] = cf + ENTITY_AWARE_COFF * reg


def _tc_finish(z, zi, zu, mi, mu):
    batch = z.shape[0]
    rows = 128
    cols = batch // rows
    out = pl.pallas_call(
        _tc_body,
        out_shape=jax.ShapeDtypeStruct((1, 1), jnp.float32),
        out_specs=pl.BlockSpec(memory_space=pltpu.SMEM),
    )(z.reshape(rows, cols), zi.reshape(rows, cols), zu.reshape(rows, cols),
      mi.reshape(rows, cols), mu.reshape(rows, cols))
    return out[0, 0]


def kernel(x, pos_neg_pair_t):
    p = pos_neg_pair_t.astype(jnp.int32)
    cols = p.T
    idx7 = jnp.concatenate(
        [cols[0:5], cols[6:8]], axis=0).reshape(-1)
    mi = cols[5].astype(jnp.float32)
    mu = cols[8].astype(jnp.float32)
    x_flat = _sc_linearize(x)
    x_lin = x_flat.reshape(x.shape)
    z, zi, zu = _sc_partials(x_lin, idx7)
    return _tc_finish(z, zi, zu, mi, mu)


# speedup vs baseline: 2.8360x; 1.5109x over previous
import functools

import jax
import jax.numpy as jnp
from jax import lax
from jax.experimental import pallas as pl
from jax.experimental.pallas import tpu as pltpu
from jax.experimental.pallas import tpu_sc as plsc

EMBED_DIM = 64
LANES = 16
N_WORKERS = 32
CHUNK = 128
ENTITY_AWARE_COFF = 0.001


TR_BLK = 512


def _tr_body(xt_hbm, out_hbm, in_v, tail_v, out_v, sem):
    n_nodes = xt_hbm.shape[1]
    n_full = n_nodes // TR_BLK
    tail = n_nodes - n_full * TR_BLK
    wid = lax.axis_index("s") * 2 + lax.axis_index("c")
    iota = lax.iota(jnp.int32, LANES)

    nblk = (n_full - 1 - wid) // N_WORKERS + 1

    rot = [(iota + s) & (LANES - 1) for s in range(LANES)]

    def transpose_groups(buf, n_groups):
        def grp(g, c):
            base = g * LANES
            colv = base + iota
            rowoff = colv * EMBED_DIM
            pairs = [rot[s] + jb * LANES
                     for jb in range(EMBED_DIM // LANES) for s in range(LANES)]
            look = 8
            vals = {k: plsc.load_gather(buf, [pairs[k], colv])
                    for k in range(look)}
            for k in range(len(pairs)):
                if k + look < len(pairs):
                    vals[k + look] = plsc.load_gather(buf, [pairs[k + look], colv])
                plsc.store_scatter(out_v, [rowoff + pairs[k]], vals[k])
            return c

        lax.fori_loop(0, n_groups, grp, 0)

    def blk(t, carry):
        i0 = pl.multiple_of((wid + t * N_WORKERS) * TR_BLK, 128)
        pltpu.async_copy(xt_hbm.at[:, pl.ds(i0, TR_BLK)], in_v, sem).wait()
        transpose_groups(in_v, TR_BLK // LANES)
        pltpu.sync_copy(out_v, out_hbm.at[pl.ds(i0 * EMBED_DIM, TR_BLK * EMBED_DIM)])
        return carry

    lax.fori_loop(0, nblk, blk, 0)

    if tail:
        @pl.when(wid == N_WORKERS - 1)
        def _():
            i0 = pl.multiple_of((wid - (N_WORKERS - 1)) + n_full * TR_BLK, 128)
            pltpu.async_copy(xt_hbm.at[:, pl.ds(i0, 128)], tail_v, sem).wait()
            transpose_groups(tail_v, tail // LANES)
            pltpu.sync_copy(
                out_v.at[pl.ds(0, tail * EMBED_DIM)],
                out_hbm.at[pl.ds(n_full * TR_BLK * EMBED_DIM, tail * EMBED_DIM)])


def _sc_linearize(x):
    n_nodes = x.shape[0]
    xt = x.T
    mesh = plsc.VectorSubcoreMesh(core_axis_name="c", subcore_axis_name="s")
    f = functools.partial(
        pl.kernel,
        out_type=jax.ShapeDtypeStruct((n_nodes * EMBED_DIM,), jnp.float32),
        mesh=mesh,
        scratch_types=[
            pltpu.VMEM((EMBED_DIM, TR_BLK), jnp.float32),
            pltpu.VMEM((EMBED_DIM, 128), jnp.float32),
            pltpu.VMEM((TR_BLK * EMBED_DIM,), jnp.float32),
            pltpu.SemaphoreType.DMA,
        ],
        compiler_params=pltpu.CompilerParams(
            use_tc_tiling_on_sc=True, needs_layout_passes=False,
            disable_bounds_checks=True),
    )(_tr_body)
    return f(xt)


def _sc_body(x_hbm, idx_hbm, pd_hbm, id_hbm, ud_hbm,
             idx_v, rows_v, pd_v, id_v, ud_v, z_v, sem):
    per_w = pd_v.shape[0]
    batch = pd_hbm.shape[0]
    n_chunks = per_w // CHUNK
    wid = lax.axis_index("s") * 2 + lax.axis_index("c")
    base_w = wid * per_w

    for k in range(7):
        pltpu.sync_copy(idx_hbm.at[pl.ds(k * batch + base_w, per_w)],
                        idx_v.at[pl.ds(k * per_w, per_w)])

    for c in range(n_chunks):
        cbase = c * CHUNK
        cps = [
            pltpu.async_copy(
                x_hbm.at[idx_v.at[pl.ds(k * per_w + cbase, CHUNK)]],
                rows_v.at[k], sem)
            for k in range(7)
        ]
        for cp in cps:
            cp.wait()

        def elem(e, carry):
            pd = None
            idp = None
            udp = None
            for j in range(EMBED_DIM // LANES):
                sl = pl.ds(j * LANES, LANES)
                uu = rows_v[0, e, sl]
                pp = rows_v[1, e, sl]
                nn = rows_v[2, e, sl]
                pe = rows_v[3, e, sl]
                ne = rows_v[4, e, sl]
                pu = rows_v[5, e, sl]
                nu = rows_v[6, e, sl]
                t_pd = uu * (pp - nn)
                a = pp - pe
                b = pp - ne
                t_id = a * a - b * b
                a2 = uu - pu
                b2 = uu - nu
                t_ud = a2 * a2 - b2 * b2
                pd = t_pd if pd is None else pd + t_pd
                idp = t_id if idp is None else idp + t_id
                udp = t_ud if udp is None else udp + t_ud
            pd_v[cbase + e, :] = pd
            id_v[cbase + e, :] = idp
            ud_v[cbase + e, :] = udp
            return carry

        lax.fori_loop(0, CHUNK, elem, 0)

    def group(g, carry):
        ids = g * LANES + lax.iota(jnp.int32, LANES)
        for t, part in enumerate((pd_v, id_v, ud_v)):
            z = None
            for l in range(LANES):
                col = plsc.load_gather(
                    part, [ids, jnp.full((LANES,), l, jnp.int32)])
                z = col if z is None else z + col
            z_v[t, pl.ds(g * LANES, LANES)] = z
        return carry

    lax.fori_loop(0, per_w // LANES, group, 0)

    pltpu.sync_copy(z_v.at[0], pd_hbm.at[pl.ds(base_w, per_w)])
    pltpu.sync_copy(z_v.at[1], id_hbm.at[pl.ds(base_w, per_w)])
    pltpu.sync_copy(z_v.at[2], ud_hbm.at[pl.ds(base_w, per_w)])


def _sc_partials(x, idx7):
    batch = idx7.shape[0] // 7
    per_w = batch // N_WORKERS
    mesh = plsc.VectorSubcoreMesh(core_axis_name="c", subcore_axis_name="s")
    out = jax.ShapeDtypeStruct((batch,), jnp.float32)
    f = functools.partial(
        pl.kernel,
        out_type=[out, out, out],
        mesh=mesh,
        scratch_types=[
            pltpu.VMEM((7 * per_w,), jnp.int32),
            pltpu.VMEM((7, CHUNK, EMBED_DIM), jnp.float32),
            pltpu.VMEM((per_w, LANES), jnp.float32),
            pltpu.VMEM((per_w, LANES), jnp.float32),
            pltpu.VMEM((per_w, LANES), jnp.float32),
            pltpu.VMEM((3, per_w), jnp.float32),
            pltpu.SemaphoreType.DMA,
        ],
        compiler_params=pltpu.CompilerParams(
            use_tc_tiling_on_sc=False, needs_layout_passes=False),
    )(_sc_body)
    return f(x, idx7)


def _tc_body(z_ref, zi_ref, zu_ref, mi_ref, mu_ref, out_ref):
    z = z_ref[...]
    zi = zi_ref[...] * mi_ref[...]
    zu = zu_ref[...] * mu_ref[...]

    def nls(t):
        mt = jnp.maximum(-t, 0.0)
        return mt + jnp.log(jnp.exp(-t - mt) + jnp.exp(-mt))

    cf = jnp.sum(nls(z))
    reg = jnp.sum(nls(zi)) + jnp.sum(nls(zu))
    out_ref[0, 0] = cf + ENTITY_AWARE_COFF * reg


def _tc_finish(z, zi, zu, mi, mu):
    batch = z.shape[0]
    rows = 128
    cols = batch // rows
    out = pl.pallas_call(
        _tc_body,
        out_shape=jax.ShapeDtypeStruct((1, 1), jnp.float32),
        out_specs=pl.BlockSpec(memory_space=pltpu.SMEM),
    )(z.reshape(rows, cols), zi.reshape(rows, cols), zu.reshape(rows, cols),
      mi.reshape(rows, cols), mu.reshape(rows, cols))
    return out[0, 0]


def kernel(x, pos_neg_pair_t):
    p = pos_neg_pair_t.astype(jnp.int32)
    cols = p.T
    idx7 = jnp.concatenate(
        [cols[0:5], cols[6:8]], axis=0).reshape(-1)
    mi = cols[5].astype(jnp.float32)
    mu = cols[8].astype(jnp.float32)
    x_flat = _sc_linearize(x)
    x_lin = x_flat.reshape(x.shape)
    z, zi, zu = _sc_partials(x_lin, idx7)
    return _tc_finish(z, zi, zu, mi, mu)


# speedup vs baseline: 2.8397x; 1.0013x over previous
import functools

import jax
import jax.numpy as jnp
from jax import lax
from jax.experimental import pallas as pl
from jax.experimental.pallas import tpu as pltpu
from jax.experimental.pallas import tpu_sc as plsc

EMBED_DIM = 64
LANES = 16
N_WORKERS = 32
CHUNK = 128
ENTITY_AWARE_COFF = 0.001


TR_BLK = 512


def _tr_body(xt_hbm, out_hbm, in_v, tail_v, out_v, sem):
    n_nodes = xt_hbm.shape[1]
    n_full = n_nodes // TR_BLK
    tail = n_nodes - n_full * TR_BLK
    wid = lax.axis_index("s") * 2 + lax.axis_index("c")
    iota = lax.iota(jnp.int32, LANES)

    nblk = (n_full - 1 - wid) // N_WORKERS + 1

    def transpose_groups(buf, n_groups):
        def grp(g, c):
            base = g * LANES
            colv = base + iota
            rowoff = colv * EMBED_DIM
            pairs = []
            for jb in range(EMBED_DIM // LANES):
                rr = iota
                for s in range(LANES):
                    pairs.append(rr + jb * LANES)
                    rr = (rr + 1) & (LANES - 1)
            look = 8
            vals = {k: plsc.load_gather(buf, [pairs[k], colv])
                    for k in range(look)}
            for k in range(len(pairs)):
                if k + look < len(pairs):
                    vals[k + look] = plsc.load_gather(buf, [pairs[k + look], colv])
                plsc.store_scatter(out_v, [rowoff + pairs[k]], vals[k])
            return c

        lax.fori_loop(0, n_groups, grp, 0)

    def blk(t, carry):
        i0 = pl.multiple_of((wid + t * N_WORKERS) * TR_BLK, 128)
        pltpu.async_copy(xt_hbm.at[:, pl.ds(i0, TR_BLK)], in_v, sem).wait()
        transpose_groups(in_v, TR_BLK // LANES)
        pltpu.sync_copy(out_v, out_hbm.at[pl.ds(i0 * EMBED_DIM, TR_BLK * EMBED_DIM)])
        return carry

    lax.fori_loop(0, nblk, blk, 0)

    if tail:
        @pl.when(wid == N_WORKERS - 1)
        def _():
            i0 = pl.multiple_of((wid - (N_WORKERS - 1)) + n_full * TR_BLK, 128)
            pltpu.async_copy(xt_hbm.at[:, pl.ds(i0, 128)], tail_v, sem).wait()
            transpose_groups(tail_v, tail // LANES)
            pltpu.sync_copy(
                out_v.at[pl.ds(0, tail * EMBED_DIM)],
                out_hbm.at[pl.ds(n_full * TR_BLK * EMBED_DIM, tail * EMBED_DIM)])


def _sc_linearize(x):
    n_nodes = x.shape[0]
    xt = x.T
    mesh = plsc.VectorSubcoreMesh(core_axis_name="c", subcore_axis_name="s")
    f = functools.partial(
        pl.kernel,
        out_type=jax.ShapeDtypeStruct((n_nodes * EMBED_DIM,), jnp.float32),
        mesh=mesh,
        scratch_types=[
            pltpu.VMEM((EMBED_DIM, TR_BLK), jnp.float32),
            pltpu.VMEM((EMBED_DIM, 128), jnp.float32),
            pltpu.VMEM((TR_BLK * EMBED_DIM,), jnp.float32),
            pltpu.SemaphoreType.DMA,
        ],
        compiler_params=pltpu.CompilerParams(
            use_tc_tiling_on_sc=True, needs_layout_passes=False,
            disable_bounds_checks=True),
    )(_tr_body)
    return f(xt)


def _sc_body(x_hbm, idx_hbm, pd_hbm, id_hbm, ud_hbm,
             idx_v, rows_v, pd_v, id_v, ud_v, z_v, sem):
    per_w = pd_v.shape[0]
    batch = pd_hbm.shape[0]
    n_chunks = per_w // CHUNK
    wid = lax.axis_index("s") * 2 + lax.axis_index("c")
    base_w = wid * per_w

    for k in range(7):
        pltpu.sync_copy(idx_hbm.at[pl.ds(k * batch + base_w, per_w)],
                        idx_v.at[pl.ds(k * per_w, per_w)])

    for c in range(n_chunks):
        cbase = c * CHUNK
        cps = [
            pltpu.async_copy(
                x_hbm.at[idx_v.at[pl.ds(k * per_w + cbase, CHUNK)]],
                rows_v.at[k], sem)
            for k in range(7)
        ]
        for cp in cps:
            cp.wait()

        def elem(e, carry):
            pd = None
            idp = None
            udp = None
            for j in range(EMBED_DIM // LANES):
                sl = pl.ds(j * LANES, LANES)
                uu = rows_v[0, e, sl]
                pp = rows_v[1, e, sl]
                nn = rows_v[2, e, sl]
                pe = rows_v[3, e, sl]
                ne = rows_v[4, e, sl]
                pu = rows_v[5, e, sl]
                nu = rows_v[6, e, sl]
                t_pd = uu * (pp - nn)
                a = pp - pe
                b = pp - ne
                t_id = a * a - b * b
                a2 = uu - pu
                b2 = uu - nu
                t_ud = a2 * a2 - b2 * b2
                pd = t_pd if pd is None else pd + t_pd
                idp = t_id if idp is None else idp + t_id
                udp = t_ud if udp is None else udp + t_ud
            pd_v[cbase + e, :] = pd
            id_v[cbase + e, :] = idp
            ud_v[cbase + e, :] = udp
            return carry

        lax.fori_loop(0, CHUNK, elem, 0)

    def group(g, carry):
        ids = g * LANES + lax.iota(jnp.int32, LANES)
        for t, part in enumerate((pd_v, id_v, ud_v)):
            z = None
            for l in range(LANES):
                col = plsc.load_gather(
                    part, [ids, jnp.full((LANES,), l, jnp.int32)])
                z = col if z is None else z + col
            z_v[t, pl.ds(g * LANES, LANES)] = z
        return carry

    lax.fori_loop(0, per_w // LANES, group, 0)

    pltpu.sync_copy(z_v.at[0], pd_hbm.at[pl.ds(base_w, per_w)])
    pltpu.sync_copy(z_v.at[1], id_hbm.at[pl.ds(base_w, per_w)])
    pltpu.sync_copy(z_v.at[2], ud_hbm.at[pl.ds(base_w, per_w)])


def _sc_partials(x, idx7):
    batch = idx7.shape[0] // 7
    per_w = batch // N_WORKERS
    mesh = plsc.VectorSubcoreMesh(core_axis_name="c", subcore_axis_name="s")
    out = jax.ShapeDtypeStruct((batch,), jnp.float32)
    f = functools.partial(
        pl.kernel,
        out_type=[out, out, out],
        mesh=mesh,
        scratch_types=[
            pltpu.VMEM((7 * per_w,), jnp.int32),
            pltpu.VMEM((7, CHUNK, EMBED_DIM), jnp.float32),
            pltpu.VMEM((per_w, LANES), jnp.float32),
            pltpu.VMEM((per_w, LANES), jnp.float32),
            pltpu.VMEM((per_w, LANES), jnp.float32),
            pltpu.VMEM((3, per_w), jnp.float32),
            pltpu.SemaphoreType.DMA,
        ],
        compiler_params=pltpu.CompilerParams(
            use_tc_tiling_on_sc=False, needs_layout_passes=False),
    )(_sc_body)
    return f(x, idx7)


def _tc_body(z_ref, zi_ref, zu_ref, mi_ref, mu_ref, out_ref):
    z = z_ref[...]
    zi = zi_ref[...] * mi_ref[...]
    zu = zu_ref[...] * mu_ref[...]

    def nls(t):
        mt = jnp.maximum(-t, 0.0)
        return mt + jnp.log(jnp.exp(-t - mt) + jnp.exp(-mt))

    cf = jnp.sum(nls(z))
    reg = jnp.sum(nls(zi)) + jnp.sum(nls(zu))
    out_ref[0, 0] = cf + ENTITY_AWARE_COFF * reg


def _tc_finish(z, zi, zu, mi, mu):
    batch = z.shape[0]
    rows = 128
    cols = batch // rows
    out = pl.pallas_call(
        _tc_body,
        out_shape=jax.ShapeDtypeStruct((1, 1), jnp.float32),
        out_specs=pl.BlockSpec(memory_space=pltpu.SMEM),
    )(z.reshape(rows, cols), zi.reshape(rows, cols), zu.reshape(rows, cols),
      mi.reshape(rows, cols), mu.reshape(rows, cols))
    return out[0, 0]


def kernel(x, pos_neg_pair_t):
    p = pos_neg_pair_t.astype(jnp.int32)
    cols = p.T
    idx7 = jnp.concatenate(
        [cols[0:5], cols[6:8]], axis=0).reshape(-1)
    mi = cols[5].astype(jnp.float32)
    mu = cols[8].astype(jnp.float32)
    x_flat = _sc_linearize(x)
    x_lin = x_flat.reshape(x.shape)
    z, zi, zu = _sc_partials(x_lin, idx7)
    return _tc_finish(z, zi, zu, mi, mu)
